# baseline (speedup 1.0000x reference)
import functools
import numpy as np
import jax
import jax.numpy as jnp
from jax import lax
from jax.experimental import pallas as pl
from jax.experimental.pallas import tpu as pltpu
from jax.experimental.pallas import tpu_sc as plsc

_N = 10000
_E = 160000
_H = 64
_NRBF = 32
_CUT = 5.0
_EP = 163840
_EB = 1024
_NB = 1000
_NCHUNK = 80

_f32 = jnp.float32
_i32 = jnp.int32


def _prep_body(az_ref, emb_ref, wlt_ref, wrt_ref, czl_ref, czr_ref):
    az = az_ref[...]
    oh = (az == lax.broadcasted_iota(_i32, (_N, 100), 1)).astype(_f32)
    zi = jnp.dot(oh, emb_ref[...], preferred_element_type=_f32)
    czl_ref[...] = jnp.dot(zi, wlt_ref[...], preferred_element_type=_f32)
    czr_ref[...] = jnp.dot(zi, wrt_ref[...], preferred_element_type=_f32)


def _prep_call(az, emb, wlt, wrt):
    return pl.pallas_call(
        _prep_body,
        out_shape=[jax.ShapeDtypeStruct((_N, _H), _f32)] * 2,
    )(az, emb, wlt, wrt)


@functools.cache
def _sc_mesh():
    return plsc.VectorSubcoreMesh(core_axis_name="c", subcore_axis_name="s")


_GB = _EP // 32 // 128


@functools.cache
def _gather_kernel():
    @functools.partial(
        pl.kernel,
        mesh=_sc_mesh(),
        compiler_params=pltpu.CompilerParams(use_tc_tiling_on_sc=False),
        out_type=[jax.ShapeDtypeStruct((_EP, _H), _f32)] * 2,
        scratch_types=[
            pltpu.VMEM((_GB, 128), _i32),
            pltpu.VMEM((_GB, 128), _i32),
            pltpu.VMEM((4, 128, _H), _f32),
            pltpu.VMEM((4, 128, _H), _f32),
            pltpu.SemaphoreType.DMA((4,)),
            pltpu.SemaphoreType.DMA((4,)),
            pltpu.SemaphoreType.DMA((4,)),
            pltpu.SemaphoreType.DMA((4,)),
        ],
    )
    def gather_k(czl_hbm, czr_hbm, src_hbm, dst_hbm, ol_hbm, or_hbm,
                 isrc, idst, bl, br, sgl, sgr, swl, swr):
        wid = lax.axis_index("s") * 2 + lax.axis_index("c")
        base = wid * (_EP // 32)
        pltpu.sync_copy(src_hbm.at[pl.ds(wid * _GB, _GB), :], isrc)
        pltpu.sync_copy(dst_hbm.at[pl.ds(wid * _GB, _GB), :], idst)

        def fetch(i, b):
            pltpu.async_copy(czl_hbm.at[isrc.at[i]], bl.at[b], sgl.at[b])
            pltpu.async_copy(czr_hbm.at[idst.at[i]], br.at[b], sgr.at[b])

        def wait_fetch(i, b):
            pltpu.make_async_copy(czl_hbm.at[isrc.at[i]], bl.at[b], sgl.at[b]).wait()
            pltpu.make_async_copy(czr_hbm.at[idst.at[i]], br.at[b], sgr.at[b]).wait()

        def put(i, b):
            e0 = base + i * 128
            pltpu.async_copy(bl.at[b], ol_hbm.at[pl.ds(e0, 128)], swl.at[b])
            pltpu.async_copy(br.at[b], or_hbm.at[pl.ds(e0, 128)], swr.at[b])

        def wait_put(i, b):
            e0 = base + i * 128
            pltpu.make_async_copy(bl.at[b], ol_hbm.at[pl.ds(e0, 128)], swl.at[b]).wait()
            pltpu.make_async_copy(br.at[b], or_hbm.at[pl.ds(e0, 128)], swr.at[b]).wait()

        for b in range(4):
            fetch(b, b)

        def step(g, carry):
            for b in range(4):
                i = 4 * g + b
                wait_fetch(i, b)
                put(i, b)
                wait_put(i, b)

                @pl.when(i + 4 < _GB)
                def _next():
                    fetch(i + 4, b)
            return carry

        lax.fori_loop(0, _GB // 4, step, 0)

    return gather_k


def _gather_call(czl, czr, src2d, dst2d):
    return _gather_kernel()(czl, czr, src2d, dst2d)


_RBF_START = float(np.exp(-_CUT))
_RBF_MEANS = np.linspace(_RBF_START, 1.0, _NRBF, dtype=np.float32)[None, :]
_RBF_BETA = float(((2.0 / _NRBF) * (1.0 - _RBF_START)) ** -2)


def _edge_body(zl_ref, zr_ref, gt_ref, meansb_ref, wwide_ref, bwide_ref,
               bijt_ref, tile_ref, sel_ref, msg_ref, rfv_ref):
    gt = gt_ref[0]
    d = gt[0:1]
    inv = 1.0 / d
    rc = jnp.where(d < _CUT, 0.5 * (jnp.cos((np.pi / _CUT) * d) + 1.0), 0.0)
    v = gt[1:4] * inv
    rv = v * rc
    ft = jnp.concatenate(
        [rc, rv,
         rv[0:1] * v[0:1], rv[1:2] * v[1:2], rv[2:3] * v[2:3],
         rv[0:1] * v[1:2], rv[0:1] * v[2:3], rv[1:2] * v[2:3]], axis=0)
    xb = jnp.broadcast_to(jnp.exp(-d), (_NRBF, _EB))
    arg = xb - meansb_ref[...]
    rfvT = jnp.exp(-_RBF_BETA * arg * arg) * jnp.broadcast_to(rc, (_NRBF, _EB))
    rfv_ref[...] = rfvT.T
    dimn = (((0,), (0,)), ((), ()))
    hi = lax.Precision.HIGHEST
    projw = lax.dot_general(rfvT, wwide_ref[...], dimn, precision=hi,
                            preferred_element_type=_f32) + bwide_ref[...]
    zsum = zl_ref[...] + zr_ref[...]
    zt = jnp.dot(zsum, tile_ref[...], precision=hi,
                 preferred_element_type=_f32) + bijt_ref[...]
    fb = lax.dot_general(ft, sel_ref[...], dimn, precision=hi,
                         preferred_element_type=_f32)
    msg_ref[...] = projw * zt * fb


def _edge_call(zl, zr, geomt, meansb, wwide, bwide, bijt, tilem, selm):
    nblk = _EP // _EB
    full = lambda s: pl.BlockSpec(s, lambda i: tuple(0 for _ in s))
    return pl.pallas_call(
        _edge_body,
        grid=(nblk,),
        in_specs=[
            pl.BlockSpec((_EB, _H), lambda i: (i, 0)),
            pl.BlockSpec((_EB, _H), lambda i: (i, 0)),
            pl.BlockSpec((1, 4, _EB), lambda i: (i, 0, 0)),
            full((_NRBF, _EB)),
            full((_NRBF, 640)), full((1, 640)), full((1, 640)),
            full((_H, 640)), full((10, 640)),
        ],
        out_specs=[
            pl.BlockSpec((_EB, 640), lambda i: (i, 0)),
            pl.BlockSpec((_EB, _NRBF), lambda i: (i, 0)),
        ],
        out_shape=[
            jax.ShapeDtypeStruct((_EP, 640), _f32),
            jax.ShapeDtypeStruct((_EP, _NRBF), _f32),
        ],
    )(zl, zr, geomt, meansb, wwide, bwide, bijt, tilem, selm)


@functools.cache
def _scatter_kernel():
    @functools.partial(
        pl.kernel,
        mesh=_sc_mesh(),
        compiler_params=pltpu.CompilerParams(use_tc_tiling_on_sc=False),
        out_type=jax.ShapeDtypeStruct((_N, 640), _f32),
        scratch_types=[
            pltpu.VMEM((_EP // 16 // 128, 128), _i32),
            pltpu.VMEM((4, 128, _NCHUNK), _f32),
            pltpu.VMEM((125, _NCHUNK), _f32),
            pltpu.VMEM_SHARED((_N, _NCHUNK), _f32),
            pltpu.SemaphoreType.DMA((4,)),
        ],
    )
    def scatter_k(msg_hbm, src_hbm, out_hbm, idx2d, mbuf, zbuf, acc_sp, sf):
        cid = lax.axis_index("c")
        sid = lax.axis_index("s")
        nblk = _EP // 16 // 128
        pltpu.sync_copy(src_hbm.at[pl.ds(sid * nblk, nblk), :], idx2d)

        def zrow(i, carry):
            def zcol(j, c2):
                zbuf[i, pl.ds(j * 16, 16)] = jnp.zeros((16,), _f32)
                return c2
            return lax.fori_loop(0, _NCHUNK // 16, zcol, carry)

        lax.fori_loop(0, 125, zrow, 0)

        for k in range(4):
            chunk = cid * 4 + k
            c0 = chunk * _NCHUNK

            def zacc(t, carry):
                pltpu.sync_copy(zbuf, acc_sp.at[pl.ds(sid * 625 + t * 125, 125), :])
                return carry

            lax.fori_loop(0, 5, zacc, 0)
            plsc.subcore_barrier()

            def fetch(i, b):
                e0 = sid * (_EP // 16) + i * 128
                pltpu.async_copy(msg_hbm.at[pl.ds(e0, 128), pl.ds(c0, _NCHUNK)],
                                 mbuf.at[b], sf.at[b])

            def wait_fetch(i, b):
                e0 = sid * (_EP // 16) + i * 128
                pltpu.make_async_copy(
                    msg_hbm.at[pl.ds(e0, 128), pl.ds(c0, _NCHUNK)],
                    mbuf.at[b], sf.at[b]).wait()

            for b in range(4):
                fetch(b, b)

            def step(g, carry):
                for b in range(4):
                    i = 4 * g + b
                    wait_fetch(i, b)
                    pltpu.sync_copy(mbuf.at[b], acc_sp.at[idx2d.at[i]], add=True)

                    @pl.when(i + 4 < nblk)
                    def _next():
                        fetch(i + 4, b)
                return carry

            lax.fori_loop(0, nblk // 4, step, 0)
            plsc.subcore_barrier()
            pltpu.sync_copy(acc_sp.at[pl.ds(sid * 625, 625), :],
                            out_hbm.at[pl.ds(sid * 625, 625), pl.ds(c0, _NCHUNK)])
            plsc.subcore_barrier()

    return scatter_k


def _scatter_call(msg, src_p):
    return _scatter_kernel()(msg, src_p)


def _slice_body(src_ref, out_ref):
    out_ref[...] = src_ref[...]


def _slice_call(rfv_p):
    sb = 1280
    return pl.pallas_call(
        _slice_body,
        grid=(_E // sb,),
        in_specs=[pl.BlockSpec((sb, _NRBF), lambda i: (i, 0))],
        out_specs=pl.BlockSpec((sb, _NRBF), lambda i: (i, 0)),
        out_shape=jax.ShapeDtypeStruct((_E, _NRBF), _f32),
    )(rfv_p)


def _node_body(acc_ref, ws0t_ref, bs0_ref, ws1t_ref, bs1_ref,
               wt0t_ref, wt1t_ref, wt2t_ref, lng_ref, lnb_ref, *out_refs):
    acc = acc_ref[...]
    cs = [acc[:, i * _H:(i + 1) * _H] for i in range(10)]
    sI, a0, a1, a2, q0, q1, q2, q3, q4, q5 = cs
    trq = q0 + q1 + q2
    norm = (3.0 * sI * sI + 2.0 * (a0 * a0 + a1 * a1 + a2 * a2)
            + q0 * q0 + q1 * q1 + q2 * q2
            + 2.0 * (q3 * q3 + q4 * q4 + q5 * q5) - trq * trq * (1.0 / 3.0))
    mu = jnp.mean(norm, axis=1, keepdims=True)
    var = jnp.mean((norm - mu) ** 2, axis=1, keepdims=True)
    ln = (norm - mu) * lax.rsqrt(var + 1e-5) * lng_ref[...] + lnb_ref[...]
    h1 = jnp.dot(ln, ws0t_ref[...], preferred_element_type=_f32) + bs0_ref[...]
    h1 = h1 * jax.nn.sigmoid(h1)
    h2 = jnp.dot(h1, ws1t_ref[...], preferred_element_type=_f32) + bs1_ref[...]
    h2 = h2 * jax.nn.sigmoid(h2)
    n0, n1, n2 = h2[:, :_H], h2[:, _H:2 * _H], h2[:, 2 * _H:]
    mm = lambda a, w: jnp.dot(a, w, preferred_element_type=_f32)
    sIp = mm(sI, wt0t_ref[...]) * n0
    a0p = mm(a0, wt1t_ref[...]) * n1
    a1p = mm(a1, wt1t_ref[...]) * n1
    a2p = mm(a2, wt1t_ref[...]) * n1
    q0p = mm(q0, wt2t_ref[...]) * n2
    q1p = mm(q1, wt2t_ref[...]) * n2
    q2p = mm(q2, wt2t_ref[...]) * n2
    q3p = mm(q3, wt2t_ref[...]) * n2
    q4p = mm(q4, wt2t_ref[...]) * n2
    q5p = mm(q5, wt2t_ref[...]) * n2
    tr3 = (q0p + q1p + q2p) * (1.0 / 3.0)
    vals = [sIp + q0p - tr3, q3p - a2p, q4p + a1p,
            q3p + a2p, sIp + q1p - tr3, q5p - a0p,
            q4p - a1p, q5p + a0p, sIp + q2p - tr3]
    for r, x in zip(out_refs, vals):
        r[...] = x


def _node_call(acc, ws0t, bs0, ws1t, bs1, wt0t, wt1t, wt2t, lng, lnb):
    nblk = _N // _NB
    full = lambda s: pl.BlockSpec(s, lambda i: (0, 0))
    return pl.pallas_call(
        _node_body,
        grid=(nblk,),
        in_specs=[
            pl.BlockSpec((_NB, 640), lambda i: (i, 0)),
            full((_H, 2 * _H)), full((1, 2 * _H)),
            full((2 * _H, 3 * _H)), full((1, 3 * _H)),
            full((_H, _H)), full((_H, _H)), full((_H, _H)),
            full((1, _H)), full((1, _H)),
        ],
        out_specs=[pl.BlockSpec((_NB, _H), lambda i: (i, 0))] * 9,
        out_shape=[jax.ShapeDtypeStruct((_N, _H), _f32)] * 9,
    )(acc, ws0t, bs0, ws1t, bs1, wt0t, wt1t, wt2t, lng, lnb)


def kernel(atomic_numbers, pair_indices, r_ij, d_ij, emb, W_ij, b_ij,
           W_I, b_I, W_A, b_A, W_S, b_S, Wt0, Wt1, Wt2, Ws0, bs0, Ws1, bs1,
           ln_g, ln_b):
    az = atomic_numbers.astype(_i32).reshape(_N, 1)
    src = pair_indices[0].astype(_i32)
    dst = pair_indices[1].astype(_i32)
    pad = _EP - _E
    src_p = jnp.concatenate([src, jnp.zeros((pad,), _i32)]).reshape(_EP // 128, 128)
    dst_p = jnp.concatenate([dst, jnp.zeros((pad,), _i32)]).reshape(_EP // 128, 128)
    geom = jnp.concatenate([d_ij.astype(_f32), r_ij.astype(_f32)], axis=1)
    geom_pad = jnp.concatenate(
        [jnp.full((pad, 1), 6.0, _f32), jnp.zeros((pad, 3), _f32)], axis=1)
    geom_p = jnp.concatenate([geom, geom_pad], axis=0)
    geomt = geom_p.T.reshape(4, _EP // _EB, _EB).transpose(1, 0, 2)

    wlt = W_ij[:, :_H].T
    wrt = W_ij[:, _H:].T
    czl, czr = _prep_call(az, emb, wlt, wrt)
    zl, zr = _gather_call(czl, czr, src_p, dst_p)

    row = lambda b: b.reshape(1, -1)
    meansb = jnp.broadcast_to(jnp.asarray(_RBF_MEANS).reshape(_NRBF, 1),
                              (_NRBF, _EB)) + jnp.zeros((_NRBF, _EB), _f32)
    wwide = jnp.concatenate([W_I.T] + [W_A.T] * 3 + [W_S.T] * 6, axis=1)
    bwide = jnp.concatenate([b_I] + [b_A] * 3 + [b_S] * 6).reshape(1, 640)
    bijt = jnp.tile(b_ij, 10).reshape(1, 640)
    tilem = jnp.tile(jnp.eye(_H, dtype=_f32), (1, 10))
    selm = jnp.asarray(np.kron(np.eye(10, dtype=np.float32),
                               np.ones((1, _H), np.float32)))
    msg, rfv_p = _edge_call(zl, zr, geomt, meansb, wwide, bwide, bijt,
                            tilem, selm)
    acc = _scatter_call(msg, src_p)

    perm = np.arange(3 * _H).reshape(_H, 3).T.reshape(-1)
    ws1p = Ws1[perm]
    bs1p = bs1[perm]
    outs = _node_call(acc, Ws0.T, row(bs0), ws1p.T, row(bs1p),
                      Wt0.T, Wt1.T, Wt2.T, row(ln_g), row(ln_b))
    X = jnp.stack(outs, axis=-1).reshape(_N, _H, 3, 3)
    rfv = _slice_call(rfv_p)
    return X, rfv[:, None, :]


# speedup vs baseline: 1.5108x; 1.5108x over previous
import functools
import numpy as np
import jax
import jax.numpy as jnp
from jax import lax
from jax.experimental import pallas as pl
from jax.experimental.pallas import tpu as pltpu
from jax.experimental.pallas import tpu_sc as plsc

_N = 10000
_E = 160000
_H = 64
_NRBF = 32
_CUT = 5.0
_EP = 163840
_EB = 1024
_NB = 1000
_NCHUNK = 80

_f32 = jnp.float32
_i32 = jnp.int32


def _prep_body(az_ref, emb_ref, wlt_ref, wrt_ref, czl_ref, czr_ref):
    az = az_ref[...]
    oh = (az == lax.broadcasted_iota(_i32, (_N, 100), 1)).astype(_f32)
    zi = jnp.dot(oh, emb_ref[...], preferred_element_type=_f32)
    czl_ref[...] = jnp.dot(zi, wlt_ref[...], preferred_element_type=_f32)
    czr_ref[...] = jnp.dot(zi, wrt_ref[...], preferred_element_type=_f32)


def _prep_call(az, emb, wlt, wrt):
    return pl.pallas_call(
        _prep_body,
        out_shape=[jax.ShapeDtypeStruct((_N, _H), _f32)] * 2,
    )(az, emb, wlt, wrt)


@functools.cache
def _sc_mesh():
    return plsc.VectorSubcoreMesh(core_axis_name="c", subcore_axis_name="s")


_GB = _EP // 32 // 128


@functools.cache
def _gather_kernel():
    @functools.partial(
        pl.kernel,
        mesh=_sc_mesh(),
        compiler_params=pltpu.CompilerParams(use_tc_tiling_on_sc=False),
        out_type=[jax.ShapeDtypeStruct((_EP, _H), _f32)] * 2,
        scratch_types=[
            pltpu.VMEM((_GB, 128), _i32),
            pltpu.VMEM((_GB, 128), _i32),
            pltpu.VMEM((4, 128, _H), _f32),
            pltpu.VMEM((4, 128, _H), _f32),
            pltpu.SemaphoreType.DMA((4,)),
            pltpu.SemaphoreType.DMA((4,)),
            pltpu.SemaphoreType.DMA((4,)),
            pltpu.SemaphoreType.DMA((4,)),
        ],
    )
    def gather_k(czl_hbm, czr_hbm, src_hbm, dst_hbm, ol_hbm, or_hbm,
                 isrc, idst, bl, br, sgl, sgr, swl, swr):
        wid = lax.axis_index("s") * 2 + lax.axis_index("c")
        base = wid * (_EP // 32)
        pltpu.sync_copy(src_hbm.at[pl.ds(wid * _GB, _GB), :], isrc)
        pltpu.sync_copy(dst_hbm.at[pl.ds(wid * _GB, _GB), :], idst)

        def fetch(i, b):
            pltpu.async_copy(czl_hbm.at[isrc.at[i]], bl.at[b], sgl.at[b])
            pltpu.async_copy(czr_hbm.at[idst.at[i]], br.at[b], sgr.at[b])

        def wait_fetch(i, b):
            pltpu.make_async_copy(czl_hbm.at[isrc.at[i]], bl.at[b], sgl.at[b]).wait()
            pltpu.make_async_copy(czr_hbm.at[idst.at[i]], br.at[b], sgr.at[b]).wait()

        def put(i, b):
            e0 = base + i * 128
            pltpu.async_copy(bl.at[b], ol_hbm.at[pl.ds(e0, 128)], swl.at[b])
            pltpu.async_copy(br.at[b], or_hbm.at[pl.ds(e0, 128)], swr.at[b])

        def wait_put(i, b):
            e0 = base + i * 128
            pltpu.make_async_copy(bl.at[b], ol_hbm.at[pl.ds(e0, 128)], swl.at[b]).wait()
            pltpu.make_async_copy(br.at[b], or_hbm.at[pl.ds(e0, 128)], swr.at[b]).wait()

        for b in range(4):
            fetch(b, b)

        def step(g, carry):
            for b in range(4):
                i = 4 * g + b
                wait_fetch(i, b)
                put(i, b)
                wait_put(i, b)

                @pl.when(i + 4 < _GB)
                def _next():
                    fetch(i + 4, b)
            return carry

        lax.fori_loop(0, _GB // 4, step, 0)

    return gather_k


def _gather_call(czl, czr, src2d, dst2d):
    return _gather_kernel()(czl, czr, src2d, dst2d)


_RBF_START = float(np.exp(-_CUT))
_RBF_MEANS = np.linspace(_RBF_START, 1.0, _NRBF, dtype=np.float32)[None, :]
_RBF_BETA = float(((2.0 / _NRBF) * (1.0 - _RBF_START)) ** -2)


def _edge_body(zl_ref, zr_ref, gt_ref, meansb_ref, wwide_ref, bwide_ref,
               bijr_ref, sel_ref, msg_ref, rfv_ref):
    gt = gt_ref[0]
    d = gt[0:1]
    inv = 1.0 / d
    rc = jnp.where(d < _CUT, 0.5 * (jnp.cos((np.pi / _CUT) * d) + 1.0), 0.0)
    v = gt[1:4] * inv
    rv = v * rc
    ft = jnp.concatenate(
        [rc, rv,
         rv[0:1] * v[0:1], rv[1:2] * v[1:2], rv[2:3] * v[2:3],
         rv[0:1] * v[1:2], rv[0:1] * v[2:3], rv[1:2] * v[2:3]], axis=0)
    xb = jnp.broadcast_to(jnp.exp(-d), (_NRBF, _EB))
    arg = xb - meansb_ref[...]
    rfvT = jnp.exp(-_RBF_BETA * arg * arg) * jnp.broadcast_to(rc, (_NRBF, _EB))
    rfv_ref[...] = rfvT.T
    dimn = (((0,), (0,)), ((), ()))
    projw = lax.dot_general(rfvT, wwide_ref[...], dimn,
                            preferred_element_type=_f32) + bwide_ref[...]
    zsb = zl_ref[...] + zr_ref[...] + bijr_ref[...]
    z2 = jnp.concatenate([zsb, zsb], axis=1)
    fth = ft.astype(jnp.bfloat16).astype(_f32)
    ftl = ft - fth
    fb = (lax.dot_general(fth, sel_ref[...], dimn, preferred_element_type=_f32)
          + lax.dot_general(ftl, sel_ref[...], dimn, preferred_element_type=_f32))
    msg = projw * fb
    for k in range(5):
        msg_ref[:, k * 128:(k + 1) * 128] = msg[:, k * 128:(k + 1) * 128] * z2


def _edge_call(zl, zr, geomt, meansb, wwide, bwide, bijr, selm):
    nblk = _EP // _EB
    full = lambda s: pl.BlockSpec(s, lambda i: tuple(0 for _ in s))
    return pl.pallas_call(
        _edge_body,
        grid=(nblk,),
        in_specs=[
            pl.BlockSpec((_EB, _H), lambda i: (i, 0)),
            pl.BlockSpec((_EB, _H), lambda i: (i, 0)),
            pl.BlockSpec((1, 4, _EB), lambda i: (i, 0, 0)),
            full((_NRBF, _EB)),
            full((_NRBF, 640)), full((1, 640)), full((1, _H)),
            full((10, 640)),
        ],
        out_specs=[
            pl.BlockSpec((_EB, 640), lambda i: (i, 0)),
            pl.BlockSpec((_EB, _NRBF), lambda i: (i, 0)),
        ],
        out_shape=[
            jax.ShapeDtypeStruct((_EP, 640), _f32),
            jax.ShapeDtypeStruct((_EP, _NRBF), _f32),
        ],
    )(zl, zr, geomt, meansb, wwide, bwide, bijr, selm)


@functools.cache
def _scatter_kernel():
    @functools.partial(
        pl.kernel,
        mesh=_sc_mesh(),
        compiler_params=pltpu.CompilerParams(use_tc_tiling_on_sc=False),
        out_type=jax.ShapeDtypeStruct((_N, 640), _f32),
        scratch_types=[
            pltpu.VMEM((_EP // 16 // 128, 128), _i32),
            pltpu.VMEM((4, 128, _NCHUNK), _f32),
            pltpu.VMEM((125, _NCHUNK), _f32),
            pltpu.VMEM_SHARED((_N, _NCHUNK), _f32),
            pltpu.SemaphoreType.DMA((4,)),
        ],
    )
    def scatter_k(msg_hbm, src_hbm, out_hbm, idx2d, mbuf, zbuf, acc_sp, sf):
        cid = lax.axis_index("c")
        sid = lax.axis_index("s")
        nblk = _EP // 16 // 128
        pltpu.sync_copy(src_hbm.at[pl.ds(sid * nblk, nblk), :], idx2d)

        def zrow(i, carry):
            def zcol(j, c2):
                zbuf[i, pl.ds(j * 16, 16)] = jnp.zeros((16,), _f32)
                return c2
            return lax.fori_loop(0, _NCHUNK // 16, zcol, carry)

        lax.fori_loop(0, 125, zrow, 0)

        for k in range(4):
            chunk = cid * 4 + k
            c0 = chunk * _NCHUNK

            def zacc(t, carry):
                pltpu.sync_copy(zbuf, acc_sp.at[pl.ds(sid * 625 + t * 125, 125), :])
                return carry

            lax.fori_loop(0, 5, zacc, 0)
            plsc.subcore_barrier()

            def fetch(i, b):
                e0 = sid * (_EP // 16) + i * 128
                pltpu.async_copy(msg_hbm.at[pl.ds(e0, 128), pl.ds(c0, _NCHUNK)],
                                 mbuf.at[b], sf.at[b])

            def wait_fetch(i, b):
                e0 = sid * (_EP // 16) + i * 128
                pltpu.make_async_copy(
                    msg_hbm.at[pl.ds(e0, 128), pl.ds(c0, _NCHUNK)],
                    mbuf.at[b], sf.at[b]).wait()

            for b in range(4):
                fetch(b, b)

            def step(g, carry):
                for b in range(4):
                    i = 4 * g + b
                    wait_fetch(i, b)
                    pltpu.sync_copy(mbuf.at[b], acc_sp.at[idx2d.at[i]], add=True)

                    @pl.when(i + 4 < nblk)
                    def _next():
                        fetch(i + 4, b)
                return carry

            lax.fori_loop(0, nblk // 4, step, 0)
            plsc.subcore_barrier()
            pltpu.sync_copy(acc_sp.at[pl.ds(sid * 625, 625), :],
                            out_hbm.at[pl.ds(sid * 625, 625), pl.ds(c0, _NCHUNK)])
            plsc.subcore_barrier()

    return scatter_k


def _scatter_call(msg, src_p):
    return _scatter_kernel()(msg, src_p)


def _slice_body(src_ref, out_ref):
    out_ref[...] = src_ref[...]


def _slice_call(rfv_p):
    sb = 1280
    return pl.pallas_call(
        _slice_body,
        grid=(_E // sb,),
        in_specs=[pl.BlockSpec((sb, _NRBF), lambda i: (i, 0))],
        out_specs=pl.BlockSpec((sb, _NRBF), lambda i: (i, 0)),
        out_shape=jax.ShapeDtypeStruct((_E, _NRBF), _f32),
    )(rfv_p)


def _node_body(acc_ref, ws0t_ref, bs0_ref, ws1t_ref, bs1_ref,
               wt0t_ref, wt1t_ref, wt2t_ref, lng_ref, lnb_ref, *out_refs):
    acc = acc_ref[...]
    cs = [acc[:, i * _H:(i + 1) * _H] for i in range(10)]
    sI, a0, a1, a2, q0, q1, q2, q3, q4, q5 = cs
    trq = q0 + q1 + q2
    norm = (3.0 * sI * sI + 2.0 * (a0 * a0 + a1 * a1 + a2 * a2)
            + q0 * q0 + q1 * q1 + q2 * q2
            + 2.0 * (q3 * q3 + q4 * q4 + q5 * q5) - trq * trq * (1.0 / 3.0))
    mu = jnp.mean(norm, axis=1, keepdims=True)
    var = jnp.mean((norm - mu) ** 2, axis=1, keepdims=True)
    ln = (norm - mu) * lax.rsqrt(var + 1e-5) * lng_ref[...] + lnb_ref[...]
    h1 = jnp.dot(ln, ws0t_ref[...], preferred_element_type=_f32) + bs0_ref[...]
    h1 = h1 * jax.nn.sigmoid(h1)
    h2 = jnp.dot(h1, ws1t_ref[...], preferred_element_type=_f32) + bs1_ref[...]
    h2 = h2 * jax.nn.sigmoid(h2)
    n0, n1, n2 = h2[:, :_H], h2[:, _H:2 * _H], h2[:, 2 * _H:]
    mm = lambda a, w: jnp.dot(a, w, preferred_element_type=_f32)
    sIp = mm(sI, wt0t_ref[...]) * n0
    a0p = mm(a0, wt1t_ref[...]) * n1
    a1p = mm(a1, wt1t_ref[...]) * n1
    a2p = mm(a2, wt1t_ref[...]) * n1
    q0p = mm(q0, wt2t_ref[...]) * n2
    q1p = mm(q1, wt2t_ref[...]) * n2
    q2p = mm(q2, wt2t_ref[...]) * n2
    q3p = mm(q3, wt2t_ref[...]) * n2
    q4p = mm(q4, wt2t_ref[...]) * n2
    q5p = mm(q5, wt2t_ref[...]) * n2
    tr3 = (q0p + q1p + q2p) * (1.0 / 3.0)
    vals = [sIp + q0p - tr3, q3p - a2p, q4p + a1p,
            q3p + a2p, sIp + q1p - tr3, q5p - a0p,
            q4p - a1p, q5p + a0p, sIp + q2p - tr3]
    for r, x in zip(out_refs, vals):
        r[...] = x


def _node_call(acc, ws0t, bs0, ws1t, bs1, wt0t, wt1t, wt2t, lng, lnb):
    nblk = _N // _NB
    full = lambda s: pl.BlockSpec(s, lambda i: (0, 0))
    return pl.pallas_call(
        _node_body,
        grid=(nblk,),
        in_specs=[
            pl.BlockSpec((_NB, 640), lambda i: (i, 0)),
            full((_H, 2 * _H)), full((1, 2 * _H)),
            full((2 * _H, 3 * _H)), full((1, 3 * _H)),
            full((_H, _H)), full((_H, _H)), full((_H, _H)),
            full((1, _H)), full((1, _H)),
        ],
        out_specs=[pl.BlockSpec((_NB, _H), lambda i: (i, 0))] * 9,
        out_shape=[jax.ShapeDtypeStruct((_N, _H), _f32)] * 9,
    )(acc, ws0t, bs0, ws1t, bs1, wt0t, wt1t, wt2t, lng, lnb)


def kernel(atomic_numbers, pair_indices, r_ij, d_ij, emb, W_ij, b_ij,
           W_I, b_I, W_A, b_A, W_S, b_S, Wt0, Wt1, Wt2, Ws0, bs0, Ws1, bs1,
           ln_g, ln_b):
    az = atomic_numbers.astype(_i32).reshape(_N, 1)
    src = pair_indices[0].astype(_i32)
    dst = pair_indices[1].astype(_i32)
    pad = _EP - _E
    src_p = jnp.concatenate([src, jnp.zeros((pad,), _i32)]).reshape(_EP // 128, 128)
    dst_p = jnp.concatenate([dst, jnp.zeros((pad,), _i32)]).reshape(_EP // 128, 128)
    geom = jnp.concatenate([d_ij.astype(_f32), r_ij.astype(_f32)], axis=1)
    geom_pad = jnp.concatenate(
        [jnp.full((pad, 1), 6.0, _f32), jnp.zeros((pad, 3), _f32)], axis=1)
    geom_p = jnp.concatenate([geom, geom_pad], axis=0)
    geomt = geom_p.T.reshape(4, _EP // _EB, _EB).transpose(1, 0, 2)

    wlt = W_ij[:, :_H].T
    wrt = W_ij[:, _H:].T
    czl, czr = _prep_call(az, emb, wlt, wrt)
    zl, zr = _gather_call(czl, czr, src_p, dst_p)

    row = lambda b: b.reshape(1, -1)
    meansb = jnp.broadcast_to(jnp.asarray(_RBF_MEANS).reshape(_NRBF, 1),
                              (_NRBF, _EB)) + jnp.zeros((_NRBF, _EB), _f32)
    wwide = jnp.concatenate([W_I.T] + [W_A.T] * 3 + [W_S.T] * 6, axis=1)
    bwide = jnp.concatenate([b_I] + [b_A] * 3 + [b_S] * 6).reshape(1, 640)
    selm = jnp.asarray(np.kron(np.eye(10, dtype=np.float32),
                               np.ones((1, _H), np.float32)))
    msg, rfv_p = _edge_call(zl, zr, geomt, meansb, wwide, bwide,
                            row(b_ij), selm)
    acc = _scatter_call(msg, src_p)

    perm = np.arange(3 * _H).reshape(_H, 3).T.reshape(-1)
    ws1p = Ws1[perm]
    bs1p = bs1[perm]
    outs = _node_call(acc, Ws0.T, row(bs0), ws1p.T, row(bs1p),
                      Wt0.T, Wt1.T, Wt2.T, row(ln_g), row(ln_b))
    X = jnp.stack(outs, axis=-1).reshape(_N, _H, 3, 3)
    rfv = _slice_call(rfv_p)
    return X, rfv[:, None, :]
